# baseline (speedup 1.0000x reference)
import functools

import jax
import jax.numpy as jnp
from jax import lax
from jax.experimental import pallas as pl
from jax.experimental.pallas import tpu as pltpu
from jax.experimental.pallas import tpu_sc as plsc

N_ROWS = 16384
N_FEAT = 2048
BLOCK_ROWS = 8


def kernel(x, indices):
    info = plsc.get_sparse_core_info()
    num_cores, num_subcores, lanes = (
        info.num_cores, info.num_subcores, info.num_lanes)
    num_workers = num_cores * num_subcores
    rows_per_worker = N_ROWS // num_workers
    num_blocks = rows_per_worker // BLOCK_ROWS
    num_pairs = num_blocks // 2
    mesh = plsc.VectorSubcoreMesh(core_axis_name="c", subcore_axis_name="s")

    @functools.partial(
        pl.kernel,
        mesh=mesh,
        compiler_params=pltpu.CompilerParams(needs_layout_passes=False),
        out_type=jax.ShapeDtypeStruct((N_ROWS, N_FEAT), jnp.float32),
        scratch_types=[
            pltpu.VMEM((N_FEAT,), jnp.int32),
            pltpu.VMEM((BLOCK_ROWS, N_FEAT), jnp.float32),
            pltpu.VMEM((BLOCK_ROWS, N_FEAT), jnp.float32),
            pltpu.VMEM((BLOCK_ROWS, N_FEAT), jnp.float32),
            pltpu.VMEM((BLOCK_ROWS, N_FEAT), jnp.float32),
            pltpu.VMEM((BLOCK_ROWS, N_FEAT), jnp.float32),
            pltpu.VMEM((BLOCK_ROWS, N_FEAT), jnp.float32),
            pltpu.SemaphoreType.DMA,
            pltpu.SemaphoreType.DMA,
            pltpu.SemaphoreType.DMA,
            pltpu.SemaphoreType.DMA,
            pltpu.SemaphoreType.DMA,
            pltpu.SemaphoreType.DMA,
        ],
    )
    def permute_rows(x_hbm, idx_hbm, out_hbm, idx_v,
                     in_v0, in_v1, in_v2, out_v0, out_v1, out_v2,
                     sem_in0, sem_in1, sem_in2, sem_out0, sem_out1,
                     sem_out2):
        in_bufs = (in_v0, in_v1, in_v2)
        out_bufs = (out_v0, out_v1, out_v2)
        in_sems = (sem_in0, sem_in1, sem_in2)
        out_sems = (sem_out0, sem_out1, sem_out2)

        wid = lax.axis_index("s") * num_cores + lax.axis_index("c")
        base = wid * rows_per_worker

        def in_block(b):
            return x_hbm.at[pl.ds(base + b * BLOCK_ROWS, BLOCK_ROWS)]

        def out_block(b):
            return out_hbm.at[pl.ds(base + b * BLOCK_ROWS, BLOCK_ROWS)]

        for j in range(3):
            pltpu.async_copy(in_block(j), in_bufs[j], in_sems[j])
        pltpu.sync_copy(idx_hbm, idx_v)

        last_col0 = N_FEAT - lanes
        num_cols = N_FEAT // lanes

        def permute_block(src, dst):
            def gather8(idxv):
                return [
                    plsc.load_gather(
                        src, [jnp.full((lanes,), r, jnp.int32), idxv])
                    for r in range(BLOCK_ROWS)
                ]

            def store8(col0, vals):
                for r in range(BLOCK_ROWS):
                    dst[r, pl.ds(col0, lanes)] = vals[r]

            @plsc.parallel_loop(0, num_cols, unroll=5)
            def _(jb):
                col0 = jb * lanes
                store8(col0, gather8(idx_v[pl.ds(col0, lanes)]))

        def do_block(b, ring):
            pltpu.make_async_copy(in_block(b), in_bufs[ring],
                                  in_sems[ring]).wait()
            @pl.when(b >= 3)
            def _():
                pltpu.make_async_copy(out_bufs[ring], out_block(b),
                                      out_sems[ring]).wait()
            permute_block(in_bufs[ring], out_bufs[ring])
            pltpu.async_copy(out_bufs[ring], out_block(b), out_sems[ring])

            @pl.when(b + 3 < num_blocks)
            def _():
                pltpu.async_copy(in_block(b + 3), in_bufs[ring],
                                 in_sems[ring])

        def group_body(g, carry):
            for j in range(3):
                do_block(g * 3 + j, j)
            return carry

        lax.fori_loop(0, (num_blocks - 1) // 3, group_body, 0)
        do_block(num_blocks - 1, 0)
        for j in range(3):
            pltpu.make_async_copy(out_bufs[j], out_block(0), out_sems[j]).wait()

    return permute_rows(x, indices)


# speedup vs baseline: 1.0048x; 1.0048x over previous
import functools

import jax
import jax.numpy as jnp
from jax import lax
from jax.experimental import pallas as pl
from jax.experimental.pallas import tpu as pltpu
from jax.experimental.pallas import tpu_sc as plsc

N_ROWS = 16384
N_FEAT = 2048
BLOCK_ROWS = 8


def kernel(x, indices):
    info = plsc.get_sparse_core_info()
    num_cores, num_subcores, lanes = (
        info.num_cores, info.num_subcores, info.num_lanes)
    num_workers = num_cores * num_subcores
    rows_per_worker = N_ROWS // num_workers
    num_blocks = rows_per_worker // BLOCK_ROWS
    mesh = plsc.VectorSubcoreMesh(core_axis_name="c", subcore_axis_name="s")

    @functools.partial(
        pl.kernel,
        mesh=mesh,
        compiler_params=pltpu.CompilerParams(needs_layout_passes=False),
        out_type=jax.ShapeDtypeStruct((N_ROWS, N_FEAT), jnp.float32),
        scratch_types=[
            pltpu.VMEM((N_FEAT,), jnp.int32),
            pltpu.VMEM((BLOCK_ROWS, N_FEAT), jnp.float32),
            pltpu.VMEM((BLOCK_ROWS, N_FEAT), jnp.float32),
            pltpu.VMEM((BLOCK_ROWS, N_FEAT), jnp.float32),
            pltpu.VMEM((BLOCK_ROWS, N_FEAT), jnp.float32),
            pltpu.VMEM((BLOCK_ROWS, N_FEAT), jnp.float32),
            pltpu.VMEM((BLOCK_ROWS, N_FEAT), jnp.float32),
            pltpu.SemaphoreType.DMA,
            pltpu.SemaphoreType.DMA,
            pltpu.SemaphoreType.DMA,
            pltpu.SemaphoreType.DMA,
            pltpu.SemaphoreType.DMA,
            pltpu.SemaphoreType.DMA,
        ],
    )
    def permute_rows(x_hbm, idx_hbm, out_hbm, idx_v,
                     in_v0, in_v1, in_v2, out_v0, out_v1, out_v2,
                     sem_in0, sem_in1, sem_in2, sem_out0, sem_out1,
                     sem_out2):
        in_bufs = (in_v0, in_v1, in_v2)
        out_bufs = (out_v0, out_v1, out_v2)
        in_sems = (sem_in0, sem_in1, sem_in2)
        out_sems = (sem_out0, sem_out1, sem_out2)

        wid = lax.axis_index("s") * num_cores + lax.axis_index("c")
        base = wid * rows_per_worker

        def in_block(b):
            return x_hbm.at[pl.ds(base + b * BLOCK_ROWS, BLOCK_ROWS)]

        def out_block(b):
            return out_hbm.at[pl.ds(base + b * BLOCK_ROWS, BLOCK_ROWS)]

        for j in range(3):
            pltpu.async_copy(in_block(j), in_bufs[j], in_sems[j])
        pltpu.sync_copy(idx_hbm, idx_v)

        num_cols = N_FEAT // lanes

        def permute_block(src, dst):
            def gather8(idxv):
                return [
                    plsc.load_gather(
                        src, [jnp.full((lanes,), r, jnp.int32), idxv])
                    for r in range(BLOCK_ROWS)
                ]

            def store8(col0, vals):
                for r in range(BLOCK_ROWS):
                    dst[r, pl.ds(col0, lanes)] = vals[r]

            @plsc.parallel_loop(0, num_cols, unroll=4)
            def _(jb):
                col0 = jb * lanes
                store8(col0, gather8(idx_v[pl.ds(col0, lanes)]))

        def do_block(b, ring):
            pltpu.make_async_copy(in_block(b), in_bufs[ring],
                                  in_sems[ring]).wait()
            @pl.when(b >= 3)
            def _():
                pltpu.make_async_copy(out_bufs[ring], out_block(b),
                                      out_sems[ring]).wait()
            permute_block(in_bufs[ring], out_bufs[ring])
            pltpu.async_copy(out_bufs[ring], out_block(b), out_sems[ring])

            @pl.when(b + 3 < num_blocks)
            def _():
                pltpu.async_copy(in_block(b + 3), in_bufs[ring],
                                 in_sems[ring])

        def group_body(g, carry):
            for j in range(3):
                do_block(g * 3 + j, j)
            return carry

        lax.fori_loop(0, (num_blocks - 1) // 3, group_body, 0)
        do_block(num_blocks - 1, 0)
        for j in range(3):
            pltpu.make_async_copy(out_bufs[j], out_block(0), out_sems[j]).wait()

    return permute_rows(x, indices)


# speedup vs baseline: 1.0061x; 1.0014x over previous
import functools

import jax
import jax.numpy as jnp
from jax import lax
from jax.experimental import pallas as pl
from jax.experimental.pallas import tpu as pltpu
from jax.experimental.pallas import tpu_sc as plsc

N_ROWS = 16384
N_FEAT = 2048
BLOCK_ROWS = 8


def kernel(x, indices):
    info = plsc.get_sparse_core_info()
    num_cores, num_subcores, lanes = (
        info.num_cores, info.num_subcores, info.num_lanes)
    num_workers = num_cores * num_subcores
    rows_per_worker = N_ROWS // num_workers
    num_blocks = rows_per_worker // BLOCK_ROWS
    mesh = plsc.VectorSubcoreMesh(core_axis_name="c", subcore_axis_name="s")

    @functools.partial(
        pl.kernel,
        mesh=mesh,
        compiler_params=pltpu.CompilerParams(needs_layout_passes=False,
                                             disable_bounds_checks=True),
        out_type=jax.ShapeDtypeStruct((N_ROWS, N_FEAT), jnp.float32),
        scratch_types=[
            pltpu.VMEM((N_FEAT,), jnp.int32),
            pltpu.VMEM((BLOCK_ROWS, N_FEAT), jnp.float32),
            pltpu.VMEM((BLOCK_ROWS, N_FEAT), jnp.float32),
            pltpu.VMEM((BLOCK_ROWS, N_FEAT), jnp.float32),
            pltpu.VMEM((BLOCK_ROWS, N_FEAT), jnp.float32),
            pltpu.VMEM((BLOCK_ROWS, N_FEAT), jnp.float32),
            pltpu.VMEM((BLOCK_ROWS, N_FEAT), jnp.float32),
            pltpu.SemaphoreType.DMA,
            pltpu.SemaphoreType.DMA,
            pltpu.SemaphoreType.DMA,
            pltpu.SemaphoreType.DMA,
            pltpu.SemaphoreType.DMA,
            pltpu.SemaphoreType.DMA,
        ],
    )
    def permute_rows(x_hbm, idx_hbm, out_hbm, idx_v,
                     in_v0, in_v1, in_v2, out_v0, out_v1, out_v2,
                     sem_in0, sem_in1, sem_in2, sem_out0, sem_out1,
                     sem_out2):
        in_bufs = (in_v0, in_v1, in_v2)
        out_bufs = (out_v0, out_v1, out_v2)
        in_sems = (sem_in0, sem_in1, sem_in2)
        out_sems = (sem_out0, sem_out1, sem_out2)

        wid = lax.axis_index("s") * num_cores + lax.axis_index("c")
        base = wid * rows_per_worker

        def in_block(b):
            return x_hbm.at[pl.ds(base + b * BLOCK_ROWS, BLOCK_ROWS)]

        def out_block(b):
            return out_hbm.at[pl.ds(base + b * BLOCK_ROWS, BLOCK_ROWS)]

        for j in range(3):
            pltpu.async_copy(in_block(j), in_bufs[j], in_sems[j])
        pltpu.sync_copy(idx_hbm, idx_v)

        num_cols = N_FEAT // lanes

        def permute_block(src, dst):
            def gather8(idxv):
                return [
                    plsc.load_gather(
                        src, [jnp.full((lanes,), r, jnp.int32), idxv])
                    for r in range(BLOCK_ROWS)
                ]

            def store8(col0, vals):
                for r in range(BLOCK_ROWS):
                    dst[r, pl.ds(col0, lanes)] = vals[r]

            @plsc.parallel_loop(0, num_cols, unroll=4)
            def _(jb):
                col0 = jb * lanes
                store8(col0, gather8(idx_v[pl.ds(col0, lanes)]))

        def do_block(b, ring):
            pltpu.make_async_copy(in_block(b), in_bufs[ring],
                                  in_sems[ring]).wait()
            @pl.when(b >= 3)
            def _():
                pltpu.make_async_copy(out_bufs[ring], out_block(b),
                                      out_sems[ring]).wait()
            permute_block(in_bufs[ring], out_bufs[ring])
            pltpu.async_copy(out_bufs[ring], out_block(b), out_sems[ring])

            @pl.when(b + 3 < num_blocks)
            def _():
                pltpu.async_copy(in_block(b + 3), in_bufs[ring],
                                 in_sems[ring])

        def group_body(g, carry):
            for j in range(3):
                do_block(g * 3 + j, j)
            return carry

        lax.fori_loop(0, (num_blocks - 1) // 3, group_body, 0)
        do_block(num_blocks - 1, 0)
        for j in range(3):
            pltpu.make_async_copy(out_bufs[j], out_block(0), out_sems[j]).wait()

    return permute_rows(x, indices)
